# Initial kernel scaffold; baseline (speedup 1.0000x reference)
import jax
import jax.numpy as jnp
from jax.experimental import pallas as pl
from jax.experimental.pallas import tpu as pltpu

_B = 8
_S = 512
_D = 256
_HD = 128
_V = 100000
_NB = 4
_ML = 16
_R = _B * _NB
_N0 = 50048
_N1 = _V - _N0
_NEG = jnp.float32(-3.0e38)
_BIGI = jnp.int32(2**30)
_PREC = jax.lax.Precision.HIGHEST


def _row_top4(scr_ref, n, c0):
    col = jax.lax.broadcasted_iota(jnp.int32, (_R, n), 1)
    vals, idxs = [], []
    for _ in range(4):
        x = scr_ref[:, :n]
        v = jnp.max(x, axis=1)
        i = jnp.min(jnp.where(x == v[:, None], col, _BIGI), axis=1)
        vals.append(v)
        idxs.append(i + c0)
        scr_ref[:, :n] = jnp.where(col == i[:, None], _NEG, x)
    return jnp.stack(vals, axis=1), jnp.stack(idxs, axis=1)


def _decode_kernel(src2d_ref, w_enc_ref, w_h0_ref, w_ih_ref, w_hh_ref,
                   w_c_ref, w_out_ref, emb_ref,
                   seq_out_ref, sco_out_ref,
                   scr_ref, e_ref, tokv_ref, toks_ref, sem0, sem1):
    f32 = jnp.float32

    eo_sum = []
    src_sum = []
    for b in range(_B):
        blk = src2d_ref[b * _S:(b + 1) * _S, :]
        eo = jnp.tanh(jnp.dot(blk, w_enc_ref[...], precision=_PREC))
        eo_sum.append(jnp.sum(eo, axis=0) * (1.0 / _S))
        src_sum.append(jnp.sum(blk, axis=0) * (1.0 / _S))
    enc_mean = jnp.stack(eo_sum, axis=0)
    src_mean = jnp.stack(src_sum, axis=0)
    h0b = jnp.tanh(jnp.dot(src_mean, w_h0_ref[...], precision=_PREC))
    ctx_b = jnp.dot(enc_mean, w_c_ref[...], precision=_PREC)

    ctx32 = jnp.broadcast_to(ctx_b[:, None, :], (_B, _NB, _HD)).reshape(_R, _HD)
    h_init = jnp.concatenate([h0b] * _NB, axis=0)

    bs_init = jnp.full((_B, _NB), -1e9, dtype=f32).at[:, 0].set(0.0)
    seq_init = jnp.zeros((_B, _NB, _ML), dtype=jnp.int32).at[:, :, 0].set(1)
    tok_init = jnp.ones((1, _R), dtype=jnp.int32)

    w_ih = w_ih_ref[...]
    w_hh = w_hh_ref[...]

    def step(t, carry):
        h, bs, seq, tok = carry

        tokv_ref[...] = tok
        cp = pltpu.make_async_copy(tokv_ref, toks_ref, sem0)
        cp.start()
        cp.wait()
        copies = []
        for i in range(_R):
            c = pltpu.make_async_copy(
                emb_ref.at[pl.ds(toks_ref[0, i], 1), :],
                e_ref.at[pl.ds(i, 1), :], sem1)
            c.start()
            copies.append(c)
        for c in copies:
            c.wait()
        e = e_ref[...]

        new_h = jnp.tanh(jnp.dot(e, w_ih, precision=_PREC)
                         + jnp.dot(h, w_hh, precision=_PREC) + ctx32)

        ms, ss, cvs, cis = [], [], [], []
        for c0, n in ((0, _N0), (_N0, _N1)):
            lg = jnp.dot(new_h, w_out_ref[:, c0:c0 + n],
                         preferred_element_type=f32, precision=_PREC)
            scr_ref[:, :n] = lg
            x = scr_ref[:, :n]
            m = jnp.max(x, axis=1)
            s = jnp.sum(jnp.exp(x - m[:, None]), axis=1)
            cv, ci = _row_top4(scr_ref, n, c0)
            ms.append(m)
            ss.append(s)
            cvs.append(cv)
            cis.append(ci)
        m_g = jnp.maximum(ms[0], ms[1])
        s_g = ss[0] * jnp.exp(ms[0] - m_g) + ss[1] * jnp.exp(ms[1] - m_g)
        log_s = jnp.log(s_g)
        cval = jnp.concatenate(cvs, axis=1)
        cidx = jnp.concatenate(cis, axis=1)

        ls = (cval - m_g[:, None]) - log_s[:, None]
        beam_of_row = jax.lax.broadcasted_iota(jnp.int32, (_R, 8), 0) % _NB
        cflat = beam_of_row * _V + cidx

        cs = ls.reshape(_B, _NB, 8) + bs[:, :, None]
        cs = cs.reshape(_B, _NB * 8)
        cf = cflat.reshape(_B, _NB * 8)

        vs, fs = [], []
        for _ in range(_NB):
            v = jnp.max(cs, axis=1)
            fi = jnp.min(jnp.where(cs == v[:, None], cf, _BIGI), axis=1)
            vs.append(v)
            fs.append(fi)
            cs = jnp.where(cf == fi[:, None], _NEG, cs)
        new_bs = jnp.stack(vs, axis=1)
        fidx = jnp.stack(fs, axis=1)
        bsel = jnp.floor((fidx.astype(f32) + 0.5) * (1.0 / _V)).astype(jnp.int32)
        ntok = fidx - bsel * _V

        nseq = jnp.zeros_like(seq)
        for j in range(_NB):
            nseq = nseq + jnp.where(bsel[:, :, None] == j,
                                    seq[:, j, :][:, None, :], 0)
        pos = jax.lax.broadcasted_iota(jnp.int32, (_B, _NB, _ML), 2)
        nseq = jnp.where(pos == t, ntok[:, :, None], nseq)

        return new_h, new_bs, nseq, ntok.reshape(1, _R)

    h, bs, seq, tok = jax.lax.fori_loop(
        1, _ML, step, (h_init, bs_init, seq_init, tok_init))

    lengths = jnp.sum((seq != 0).astype(f32), axis=-1)
    fsc = bs / ((lengths + 1.0) * 0.5)
    seq_out_ref[...] = seq[:, 0, :]
    sco_out_ref[...] = fsc[:, 0].reshape(1, _B)


def kernel(src, src_lengths, W_enc, W_h0, emb, W_ih, W_hh, W_c, W_out):
    del src_lengths
    src2d = src.reshape(_B * _S, _D)
    vspec = pl.BlockSpec(memory_space=pltpu.VMEM)
    seq, sco = pl.pallas_call(
        _decode_kernel,
        out_shape=[jax.ShapeDtypeStruct((_B, _ML), jnp.int32),
                   jax.ShapeDtypeStruct((1, _B), jnp.float32)],
        in_specs=[vspec] * 7 + [pl.BlockSpec(memory_space=pltpu.ANY)],
        out_specs=[vspec, vspec],
        scratch_shapes=[
            pltpu.VMEM((_R, _N0), jnp.float32),
            pltpu.VMEM((_R, _HD), jnp.float32),
            pltpu.VMEM((1, _R), jnp.int32),
            pltpu.SMEM((1, _R), jnp.int32),
            pltpu.SemaphoreType.DMA,
            pltpu.SemaphoreType.DMA,
        ],
        compiler_params=pltpu.CompilerParams(
            vmem_limit_bytes=64 * 1024 * 1024),
    )(src2d, W_enc, W_h0, W_ih, W_hh, W_c, W_out, emb)
    return seq, sco.reshape(_B)


# speedup vs baseline: 4.2025x; 4.2025x over previous
import jax
import jax.numpy as jnp
from jax.experimental import pallas as pl
from jax.experimental.pallas import tpu as pltpu

_B = 8
_S = 512
_D = 256
_HD = 128
_V = 100000
_NB = 4
_ML = 16
_R = _B * _NB
_NVR = 782
_TAIL = _V - (_NVR - 1) * 128
_NEG = -3.0e38
_BIGI = 2**30


def _xla_lane_sum(acc):
    u = acc[:, 0:8]
    for j in range(1, 16):
        u = u + acc[:, 8 * j:8 * j + 8]
    t = [u[:, i:i + 1] for i in range(8)]
    return (((t[0] + t[4]) + (t[2] + t[6]))
            + ((t[1] + t[5]) + (t[3] + t[7])))


def _encoder_kernel(src2d_ref, w_enc_ref, w_h0_ref, w_c_ref,
                    ctx_ref, h0_ref):
    eo_sum = []
    src_sum = []
    for b in range(_B):
        blk = src2d_ref[b * _S:(b + 1) * _S, :]
        eo = jnp.tanh(jnp.dot(blk, w_enc_ref[...]))
        eo_sum.append(jnp.sum(eo, axis=0) * (1.0 / _S))
        src_sum.append(jnp.sum(blk, axis=0) * (1.0 / _S))
    enc_mean = jnp.stack(eo_sum, axis=0)
    src_mean = jnp.stack(src_sum, axis=0)
    h0_ref[...] = jnp.tanh(jnp.dot(src_mean, w_h0_ref[...]))
    ctx_ref[...] = jnp.dot(enc_mean, w_c_ref[...])


_WBLK = 6272
_NWB = 15
_WTOFF = _NWB * _WBLK
_WTAIL = _V - _WTOFF
_NBUF = 3
assert _NWB % _NBUF == 0


def _decode_kernel(w_ih_ref, w_hh_ref, w_out_ref, emb_ref, ctx_ref, h0_ref,
                   seq_out_ref, sco_out_ref,
                   scr_ref, e_ref, tokv_ref, toks_ref, wbuf_ref, wtail_ref,
                   sem0, sem1, sem2, wsems):
    f32 = jnp.float32

    def w_copy(k):
        return pltpu.make_async_copy(
            w_out_ref.at[:, pl.ds(k * _WBLK, _WBLK)],
            wbuf_ref.at[k % _NBUF],
            wsems.at[k % _NBUF])
    h0b = h0_ref[...]
    ctx_b = ctx_ref[...]

    ctx32 = jnp.broadcast_to(
        ctx_b[:, None, :], (_B, _NB, _HD)).reshape(_R, _HD)
    h_init = jnp.concatenate([h0b] * _NB, axis=0)

    slot = jax.lax.broadcasted_iota(jnp.int32, (_B, _NB), 1)
    bs_init = jnp.where(slot == 0, 0.0, -1e9).astype(f32)
    pos0 = jax.lax.broadcasted_iota(jnp.int32, (_B, _NB, _ML), 2)
    seq_init = jnp.where(pos0 == 0, 1, 0).astype(jnp.int32)
    tok_init = jnp.ones((_B, _NB), dtype=jnp.int32)

    w_ih = w_ih_ref[...]
    w_hh = w_hh_ref[...]

    for k in range(_NBUF):
        w_copy(k).start()
    tail_cp = pltpu.make_async_copy(
        w_out_ref.at[:, pl.ds(_WTOFF, _WTAIL)], wtail_ref, sem2)
    tail_cp.start()
    tail_cp.wait()

    def step(t, carry):
        h, bs, seq, tok = carry

        tokv_ref[...] = tok
        cp = pltpu.make_async_copy(tokv_ref, toks_ref, sem0)
        cp.start()
        cp.wait()
        copies = []
        for i in range(_R):
            c = pltpu.make_async_copy(
                emb_ref.at[pl.ds(toks_ref[i // _NB, i % _NB], 1), :],
                e_ref.at[pl.ds(i, 1), :], sem1)
            c.start()
            copies.append(c)
        for c in copies:
            c.wait()
        e = e_ref[...]

        new_h = jnp.tanh(jnp.dot(e, w_ih) + jnp.dot(h, w_hh) + ctx32)

        for k in range(_NWB):
            w_copy(k).wait()
            scr_ref[:, k * _WBLK:(k + 1) * _WBLK] = jnp.dot(
                new_h, wbuf_ref[k % _NBUF], preferred_element_type=f32)
            w_copy((k + _NBUF) % _NWB).start()
        scr_ref[:, _WTOFF:_V] = jnp.dot(
            new_h, wtail_ref[...], preferred_element_type=f32)
        m = jnp.max(scr_ref[...], axis=1)

        col = jax.lax.broadcasted_iota(jnp.int32, (_R, _V), 1)
        vals, idxs = [], []
        excl = None
        for _ in range(4):
            x = scr_ref[...]
            if excl is None:
                v = jnp.max(x, axis=1)
                i = jnp.min(jnp.where(x == v[:, None], col, _BIGI), axis=1)
            else:
                v = jnp.max(jnp.where(excl, _NEG, x), axis=1)
                i = jnp.min(jnp.where((x == v[:, None]) & (~excl),
                                      col, _BIGI), axis=1)
            vals.append(v)
            idxs.append(i)
            hit = col == i[:, None]
            excl = hit if excl is None else (excl | hit)
        cval = jnp.stack(vals, axis=1)
        cidx = jnp.stack(idxs, axis=1)

        acc = jnp.zeros((_R, 128), f32)
        for j in range(_NVR - 1):
            acc = acc + jnp.exp(
                scr_ref[:, j * 128:(j + 1) * 128] - m[:, None])
        tail_ex = jnp.exp(
            scr_ref[:, (_NVR - 1) * 128:_V] - m[:, None])
        acc = acc + jnp.concatenate(
            [tail_ex, jnp.zeros((_R, 128 - _TAIL), f32)], axis=1)
        s = _xla_lane_sum(acc)
        log_s = jnp.log(s)

        ls = (cval - m[:, None]) - log_s
        beam_of_row = jax.lax.broadcasted_iota(jnp.int32, (_R, 4), 0) % _NB
        cflat = beam_of_row * _V + cidx

        cs = ls.reshape(_B, _NB, 4) + bs[:, :, None]
        cf = cflat.reshape(_B, _NB, 4)

        vs, fs = [], []
        for _ in range(_NB):
            v = jnp.max(cs, axis=(1, 2))
            fi = jnp.min(jnp.where(cs == v[:, None, None], cf, _BIGI),
                         axis=(1, 2))
            vs.append(v)
            fs.append(fi)
            cs = jnp.where(cf == fi[:, None, None], _NEG, cs)
        new_bs = jnp.stack(vs, axis=1)
        fidx = jnp.stack(fs, axis=1)
        bsel = jnp.floor(
            (fidx.astype(f32) + 0.5) * (1.0 / _V)).astype(jnp.int32)
        ntok = fidx - bsel * _V

        nseq = jnp.zeros_like(seq)
        for j in range(_NB):
            nseq = nseq + jnp.where(bsel[:, :, None] == j,
                                    seq[:, j, :][:, None, :], 0)
        pos = jax.lax.broadcasted_iota(jnp.int32, (_B, _NB, _ML), 2)
        nseq = jnp.where(pos == t, ntok[:, :, None], nseq)

        return new_h, new_bs, nseq, ntok

    h, bs, seq, tok = jax.lax.fori_loop(
        1, _ML, step, (h_init, bs_init, seq_init, tok_init))

    for k in range(_NBUF):
        w_copy(k).wait()

    lengths = jnp.sum((seq != 0).astype(f32), axis=-1)
    fsc = bs / ((lengths + 1.0) * 0.5)
    seq_out_ref[...] = seq[:, 0, :]
    sco_out_ref[...] = fsc[:, 0:1]


def kernel(src, src_lengths, W_enc, W_h0, emb, W_ih, W_hh, W_c, W_out):
    del src_lengths
    src2d = src.reshape(_B * _S, _D)
    vspec = pl.BlockSpec(memory_space=pltpu.VMEM)

    ctx_b, h0b = pl.pallas_call(
        _encoder_kernel,
        out_shape=[jax.ShapeDtypeStruct((_B, _HD), jnp.float32),
                   jax.ShapeDtypeStruct((_B, _HD), jnp.float32)],
        in_specs=[vspec] * 4,
        out_specs=[vspec, vspec],
    )(src2d, W_enc, W_h0, W_c)

    seq, sco = pl.pallas_call(
        _decode_kernel,
        out_shape=[jax.ShapeDtypeStruct((_B, _ML), jnp.int32),
                   jax.ShapeDtypeStruct((_B, 1), jnp.float32)],
        in_specs=[vspec, vspec, pl.BlockSpec(memory_space=pl.ANY),
                  pl.BlockSpec(memory_space=pl.ANY), vspec, vspec],
        out_specs=[vspec, vspec],
        scratch_shapes=[
            pltpu.VMEM((_R, _V), jnp.float32),
            pltpu.VMEM((_R, _HD), jnp.float32),
            pltpu.VMEM((_B, _NB), jnp.int32),
            pltpu.SMEM((_B, _NB), jnp.int32),
            pltpu.VMEM((_NBUF, _HD, _WBLK), jnp.float32),
            pltpu.VMEM((_HD, _WTAIL), jnp.float32),
            pltpu.SemaphoreType.DMA,
            pltpu.SemaphoreType.DMA,
            pltpu.SemaphoreType.DMA,
            pltpu.SemaphoreType.DMA((_NBUF,)),
        ],
        compiler_params=pltpu.CompilerParams(
            vmem_limit_bytes=67108864),
    )(W_ih, W_hh, W_out, emb, ctx_b, h0b)
    return seq, sco.reshape(_B)


# speedup vs baseline: 8.6380x; 2.0554x over previous
import jax
import jax.numpy as jnp
from jax.experimental import pallas as pl
from jax.experimental.pallas import tpu as pltpu

_B = 8
_S = 512
_D = 256
_HD = 128
_V = 100000
_NB = 4
_ML = 16
_R = _B * _NB
_NVR = 782
_TAIL = _V - (_NVR - 1) * 128
_NEG = -3.0e38
_BIGI = 2**30

_BW = 3200
_MBLOCKS = []
_o = 0
while _o < _V:
    _MBLOCKS.append((_o, min(_BW, _V - _o)))
    _o += _BW


def _xla_lane_sum(acc):
    u = acc[:, 0:8]
    for j in range(1, 16):
        u = u + acc[:, 8 * j:8 * j + 8]
    t = [u[:, i:i + 1] for i in range(8)]
    return (((t[0] + t[4]) + (t[2] + t[6]))
            + ((t[1] + t[5]) + (t[3] + t[7])))


def _encoder_kernel(src2d_ref, w_enc_ref, w_h0_ref, w_c_ref,
                    ctx_ref, h0_ref):
    eo_sum = []
    src_sum = []
    for b in range(_B):
        blk = src2d_ref[b * _S:(b + 1) * _S, :]
        eo = jnp.tanh(jnp.dot(blk, w_enc_ref[...]))
        eo_sum.append(jnp.sum(eo, axis=0) * (1.0 / _S))
        src_sum.append(jnp.sum(blk, axis=0) * (1.0 / _S))
    enc_mean = jnp.stack(eo_sum, axis=0)
    src_mean = jnp.stack(src_sum, axis=0)
    h0_ref[...] = jnp.tanh(jnp.dot(src_mean, w_h0_ref[...]))
    ctx_ref[...] = jnp.dot(enc_mean, w_c_ref[...])


def _decode_kernel(w_ih_ref, w_hh_ref, w_out_ref, emb_ref, ctx_ref, h0_ref,
                   seq_out_ref, sco_out_ref,
                   e_ref, tokv_ref, toks_ref, sem0, sem1):
    f32 = jnp.float32
    h0b = h0_ref[...]
    ctx_b = ctx_ref[...]

    ctx32 = jnp.broadcast_to(
        ctx_b[:, None, :], (_B, _NB, _HD)).reshape(_R, _HD)
    h_init = jnp.concatenate([h0b] * _NB, axis=0)

    slot = jax.lax.broadcasted_iota(jnp.int32, (_B, _NB), 1)
    bs_init = jnp.where(slot == 0, 0.0, -1e9).astype(f32)
    pos0 = jax.lax.broadcasted_iota(jnp.int32, (_B, _NB, _ML), 2)
    seq_init = jnp.where(pos0 == 0, 1, 0).astype(jnp.int32)
    tok_init = jnp.ones((_B, _NB), dtype=jnp.int32)

    w_ih = w_ih_ref[...]
    w_hh = w_hh_ref[...]
    lane = jax.lax.broadcasted_iota(jnp.int32, (_R, 128), 1)

    def step(t, carry):
        h, bs, seq, tok = carry

        tokv_ref[...] = tok
        cp = pltpu.make_async_copy(tokv_ref, toks_ref, sem0)
        cp.start()
        cp.wait()
        copies = []
        for i in range(_R):
            c = pltpu.make_async_copy(
                emb_ref.at[pl.ds(toks_ref[i // _NB, i % _NB], 1), :],
                e_ref.at[pl.ds(i, 1), :], sem1)
            c.start()
            copies.append(c)
        for c in copies:
            c.wait()
        e = e_ref[...]

        new_h = jnp.tanh(jnp.dot(e, w_ih) + jnp.dot(h, w_hh) + ctx32)

        m1 = jnp.full((_R, 128), _NEG, f32)
        m2 = jnp.full((_R, 128), _NEG, f32)
        m3 = jnp.full((_R, 128), _NEG, f32)
        m4 = jnp.full((_R, 128), _NEG, f32)
        zi = jnp.zeros((_R, 128), jnp.int32)
        a1, a2, a3, a4 = zi, zi, zi, zi
        for off, w in _MBLOCKS:
            lg = jnp.dot(new_h, w_out_ref[:, off:off + w],
                         preferred_element_type=f32)
            for j in range(0, w, 128):
                wj = min(128, w - j)
                x = lg[:, j:j + wj]
                if wj < 128:
                    x = jnp.concatenate(
                        [x, jnp.full((_R, 128 - wj), _NEG, f32)], axis=1)
                c = jnp.full((_R, 128), (off + j) // 128, jnp.int32)
                g1 = x > m1
                g2 = x > m2
                g3 = x > m3
                g4 = x > m4
                m4 = jnp.where(g4, jnp.where(g3, m3, x), m4)
                a4 = jnp.where(g4, jnp.where(g3, a3, c), a4)
                m3 = jnp.where(g3, jnp.where(g2, m2, x), m3)
                a3 = jnp.where(g3, jnp.where(g2, a2, c), a3)
                m2 = jnp.where(g2, jnp.where(g1, m1, x), m2)
                a2 = jnp.where(g2, jnp.where(g1, a1, c), a2)
                m1 = jnp.where(g1, x, m1)
                a1 = jnp.where(g1, c, a1)

        cm = jnp.concatenate([m1, m2, m3, m4], axis=1)
        gcol = jnp.concatenate(
            [a1 * 128 + lane, a2 * 128 + lane,
             a3 * 128 + lane, a4 * 128 + lane], axis=1)
        vals, idxs = [], []
        for _ in range(4):
            v = jnp.max(cm, axis=1)
            i = jnp.min(jnp.where(cm == v[:, None], gcol, _BIGI), axis=1)
            vals.append(v)
            idxs.append(i)
            cm = jnp.where(gcol == i[:, None], _NEG, cm)
        cval = jnp.stack(vals, axis=1)
        cidx = jnp.stack(idxs, axis=1)
        m = vals[0]

        acc = jnp.zeros((_R, 128), f32)
        for off, w in _MBLOCKS:
            lg = jnp.dot(new_h, w_out_ref[:, off:off + w],
                         preferred_element_type=f32)
            for j in range(0, w, 128):
                wj = min(128, w - j)
                ex = jnp.exp(lg[:, j:j + wj] - m[:, None])
                if wj < 128:
                    ex = jnp.concatenate(
                        [ex, jnp.zeros((_R, 128 - wj), f32)], axis=1)
                acc = acc + ex
        s = _xla_lane_sum(acc)
        log_s = jnp.log(s)

        ls = (cval - m[:, None]) - log_s
        beam_of_row = jax.lax.broadcasted_iota(jnp.int32, (_R, 4), 0) % _NB
        cflat = beam_of_row * _V + cidx

        cs = ls.reshape(_B, _NB, 4) + bs[:, :, None]
        cf = cflat.reshape(_B, _NB, 4)

        vs, fs = [], []
        for _ in range(_NB):
            v = jnp.max(cs, axis=(1, 2))
            fi = jnp.min(jnp.where(cs == v[:, None, None], cf, _BIGI),
                         axis=(1, 2))
            vs.append(v)
            fs.append(fi)
            cs = jnp.where(cf == fi[:, None, None], _NEG, cs)
        new_bs = jnp.stack(vs, axis=1)
        fidx = jnp.stack(fs, axis=1)
        bsel = jnp.floor(
            (fidx.astype(f32) + 0.5) * (1.0 / _V)).astype(jnp.int32)
        ntok = fidx - bsel * _V

        nseq = jnp.zeros_like(seq)
        for j in range(_NB):
            nseq = nseq + jnp.where(bsel[:, :, None] == j,
                                    seq[:, j, :][:, None, :], 0)
        pos = jax.lax.broadcasted_iota(jnp.int32, (_B, _NB, _ML), 2)
        nseq = jnp.where(pos == t, ntok[:, :, None], nseq)

        return new_h, new_bs, nseq, ntok

    h, bs, seq, tok = jax.lax.fori_loop(
        1, _ML, step, (h_init, bs_init, seq_init, tok_init))

    lengths = jnp.sum((seq != 0).astype(f32), axis=-1)
    fsc = bs / ((lengths + 1.0) * 0.5)
    seq_out_ref[...] = seq[:, 0, :]
    sco_out_ref[...] = fsc[:, 0:1]


def kernel(src, src_lengths, W_enc, W_h0, emb, W_ih, W_hh, W_c, W_out):
    del src_lengths
    src2d = src.reshape(_B * _S, _D)
    vspec = pl.BlockSpec(memory_space=pltpu.VMEM)

    ctx_b, h0b = pl.pallas_call(
        _encoder_kernel,
        out_shape=[jax.ShapeDtypeStruct((_B, _HD), jnp.float32),
                   jax.ShapeDtypeStruct((_B, _HD), jnp.float32)],
        in_specs=[vspec] * 4,
        out_specs=[vspec, vspec],
    )(src2d, W_enc, W_h0, W_c)

    seq, sco = pl.pallas_call(
        _decode_kernel,
        out_shape=[jax.ShapeDtypeStruct((_B, _ML), jnp.int32),
                   jax.ShapeDtypeStruct((_B, 1), jnp.float32)],
        in_specs=[vspec, vspec, vspec,
                  pl.BlockSpec(memory_space=pl.ANY), vspec, vspec],
        out_specs=[vspec, vspec],
        scratch_shapes=[
            pltpu.VMEM((_R, _HD), jnp.float32),
            pltpu.VMEM((_B, _NB), jnp.int32),
            pltpu.SMEM((_B, _NB), jnp.int32),
            pltpu.SemaphoreType.DMA,
            pltpu.SemaphoreType.DMA,
        ],
        compiler_params=pltpu.CompilerParams(
            vmem_limit_bytes=67108864),
    )(W_ih, W_hh, W_out, emb, ctx_b, h0b)
    return seq, sco.reshape(_B)
